# Initial kernel scaffold; baseline (speedup 1.0000x reference)
import jax
import jax.numpy as jnp
from jax.experimental import pallas as pl


def kernel(t, z):
    raise NotImplementedError("write your pallas kernel here")


# speedup vs baseline: 5.7573x; 5.7573x over previous
import functools

import jax
import jax.numpy as jnp
from jax import lax
from jax.experimental import pallas as pl

NUM_INTERVALS = 128
MAX_TIME = 1.0
D_FEAT = 128

ROWS_PER_BLOCK = 512


def _body(t_ref, z_ref, tri_ref, ind_ref, dt_ref, dtind_ref, tauind_ref,
          taunext_ref, z0_ref):
    z = z_ref[...]
    v = z[:, D_FEAT:]
    z0_ref[...] = z[:, :D_FEAT]

    m = jnp.max(v, axis=1, keepdims=True)
    e = jnp.exp(v - m)
    s = jnp.sum(e, axis=1, keepdims=True)
    dt = e / s
    dt_ref[...] = dt

    tau = lax.dot_general(dt, tri_ref[...], (((1,), (0,)), ((), ())),
                          precision=lax.Precision.HIGHEST,
                          preferred_element_type=jnp.float32)

    t = t_ref[...]
    lanes = lax.broadcasted_iota(jnp.int32, tau.shape, 1)
    below = (tau < t) & (lanes < NUM_INTERVALS - 1)
    ind = jnp.sum(below.astype(jnp.int32), axis=1, keepdims=True)
    ind_ref[...] = ind

    onehot = (lanes == ind).astype(jnp.float32)
    tau_next = jnp.sum(tau * onehot, axis=1, keepdims=True)
    dt_ind = jnp.sum(dt * onehot, axis=1, keepdims=True)
    taunext_ref[...] = tau_next
    dtind_ref[...] = dt_ind
    tauind_ref[...] = tau_next - dt_ind


@jax.jit
def kernel(t, z):
    n = t.shape[0]
    r = ROWS_PER_BLOCK
    grid = (n // r,)
    tri = jnp.triu(jnp.ones((NUM_INTERVALS, NUM_INTERVALS), jnp.float32))

    out_shapes = (
        jax.ShapeDtypeStruct((n, 1), jnp.int32),
        jax.ShapeDtypeStruct((n, NUM_INTERVALS), jnp.float32),
        jax.ShapeDtypeStruct((n, 1), jnp.float32),
        jax.ShapeDtypeStruct((n, 1), jnp.float32),
        jax.ShapeDtypeStruct((n, 1), jnp.float32),
        jax.ShapeDtypeStruct((n, D_FEAT), jnp.float32),
    )
    col = lambda i: (i, 0)
    out_specs = (
        pl.BlockSpec((r, 1), col),
        pl.BlockSpec((r, NUM_INTERVALS), col),
        pl.BlockSpec((r, 1), col),
        pl.BlockSpec((r, 1), col),
        pl.BlockSpec((r, 1), col),
        pl.BlockSpec((r, D_FEAT), col),
    )
    in_specs = [
        pl.BlockSpec((r, 1), col),
        pl.BlockSpec((r, D_FEAT + NUM_INTERVALS), col),
        pl.BlockSpec((NUM_INTERVALS, NUM_INTERVALS), lambda i: (0, 0)),
    ]

    ind, dt, dt_ind, tau_ind, tau_next, z0 = pl.pallas_call(
        _body,
        grid=grid,
        in_specs=in_specs,
        out_specs=out_specs,
        out_shape=out_shapes,
    )(t.reshape(n, 1), z, tri)

    return (ind.reshape(n), dt, dt_ind.reshape(n), tau_ind.reshape(n),
            tau_next.reshape(n), z0)


# speedup vs baseline: 7.6995x; 1.3374x over previous
import functools

import jax
import jax.numpy as jnp
import numpy as np
from jax import lax
from jax.experimental import pallas as pl
from jax.experimental.pallas import tpu as pltpu
from jax.experimental.pallas import tpu_sc as plsc

NUM_INTERVALS = 128
MAX_TIME = 1.0
D_FEAT = 128
N_ROWS = 131072

NC = 2
NS = 16
L = 16
NV = NUM_INTERVALS // L

CHUNK = 128


def _sc_body(t_hbm, z_hbm, ind_hbm, dt_hbm, dtind_hbm, tauind_hbm,
             taunext_hbm, vin, tin, dtbuf, taubuf, indbuf, dtindbuf,
             tauindbuf, taunextbuf):
    c = lax.axis_index("c")
    s = lax.axis_index("s")
    wid = c * NS + s
    rows_per = N_ROWS // (NC * NS)
    base = wid * rows_per
    lane = lax.iota(jnp.int32, L)

    def chunk_body(ci, carry):
        row0 = base + ci * CHUNK
        pltpu.sync_copy(z_hbm.at[pl.ds(row0, CHUNK), pl.ds(D_FEAT, NUM_INTERVALS)],
                        vin)
        pltpu.sync_copy(t_hbm.at[pl.ds(row0, CHUNK)], tin)

        def group_body(g, gcarry):
            ind_acc = jnp.zeros((L,), jnp.int32)
            t16 = tin[pl.ds(g * L, L)]
            for j in range(L):
                r = g * L + j
                tj = t16[j]
                e = []
                ps = []
                for i in range(NV):
                    ei = jnp.exp(vin[r, pl.ds(i * L, L)])
                    e.append(ei)
                    ps.append(jnp.sum(ei))
                prefix = []
                tot = np.float32(0.0)
                for i in range(NV):
                    prefix.append(tot)
                    tot = tot + ps[i]
                inv = 1.0 / jnp.broadcast_to(tot, (L,))
                for i in range(NV):
                    dti = e[i] * inv
                    dtbuf[r, pl.ds(i * L, L)] = dti
                    taui = (plsc.cumsum(e[i]) + prefix[i]) * inv
                    taubuf[j, pl.ds(i * L, L)] = taui
                    m = taui < tj
                    if i == NV - 1:
                        m = m & (lane < L - 1)
                    pc = plsc.all_reduce_population_count(m)
                    ind_acc = ind_acc + jnp.where(lane == j, pc, 0)
            rows16 = g * L + lane
            dtind16 = plsc.load_gather(dtbuf, [rows16, ind_acc])
            taunext16 = plsc.load_gather(taubuf, [lane, ind_acc])
            indbuf[pl.ds(g * L, L)] = ind_acc
            dtindbuf[pl.ds(g * L, L)] = dtind16
            taunextbuf[pl.ds(g * L, L)] = taunext16
            tauindbuf[pl.ds(g * L, L)] = taunext16 - dtind16
            return gcarry

        lax.fori_loop(0, CHUNK // L, group_body, 0)

        pltpu.sync_copy(dtbuf, dt_hbm.at[pl.ds(row0, CHUNK)])
        pltpu.sync_copy(indbuf, ind_hbm.at[pl.ds(row0, CHUNK)])
        pltpu.sync_copy(dtindbuf, dtind_hbm.at[pl.ds(row0, CHUNK)])
        pltpu.sync_copy(tauindbuf, tauind_hbm.at[pl.ds(row0, CHUNK)])
        pltpu.sync_copy(taunextbuf, taunext_hbm.at[pl.ds(row0, CHUNK)])
        return carry

    lax.fori_loop(0, rows_per // CHUNK, chunk_body, 0)


@jax.jit
def kernel(t, z):
    n = t.shape[0]
    mesh = plsc.VectorSubcoreMesh(core_axis_name="c", subcore_axis_name="s")
    out_type = (
        jax.ShapeDtypeStruct((n,), jnp.int32),
        jax.ShapeDtypeStruct((n, NUM_INTERVALS), jnp.float32),
        jax.ShapeDtypeStruct((n,), jnp.float32),
        jax.ShapeDtypeStruct((n,), jnp.float32),
        jax.ShapeDtypeStruct((n,), jnp.float32),
    )
    scratch = [
        pltpu.VMEM((CHUNK, NUM_INTERVALS), jnp.float32),
        pltpu.VMEM((CHUNK,), jnp.float32),
        pltpu.VMEM((CHUNK, NUM_INTERVALS), jnp.float32),
        pltpu.VMEM((L, NUM_INTERVALS), jnp.float32),
        pltpu.VMEM((CHUNK,), jnp.int32),
        pltpu.VMEM((CHUNK,), jnp.float32),
        pltpu.VMEM((CHUNK,), jnp.float32),
        pltpu.VMEM((CHUNK,), jnp.float32),
    ]
    ind, dt, dt_ind, tau_ind, tau_next = pl.kernel(
        _sc_body,
        out_type=out_type,
        mesh=mesh,
        scratch_types=scratch,
        compiler_params=pltpu.CompilerParams(needs_layout_passes=False),
    )(t, z)
    z0 = z[:, :D_FEAT]
    return (ind, dt, dt_ind, tau_ind, tau_next, z0)


# speedup vs baseline: 10.7037x; 1.3902x over previous
import functools

import jax
import jax.numpy as jnp
import numpy as np
from jax import lax
from jax.experimental import pallas as pl
from jax.experimental.pallas import tpu as pltpu
from jax.experimental.pallas import tpu_sc as plsc

NUM_INTERVALS = 128
MAX_TIME = 1.0
D_FEAT = 128
N_ROWS = 131072

NC = 2
NS = 16
L = 16
NV = NUM_INTERVALS // L

CHUNK = 128
NCHUNKS = N_ROWS // (NC * NS) // CHUNK


def _sc_body(t_hbm, z_hbm, ind_hbm, dt_hbm, dtind_hbm, tauind_hbm,
             taunext_hbm, vin, tin, dtbuf, taubuf, indbuf, dtindbuf,
             tauindbuf, taunextbuf, insem, outsem):
    c = lax.axis_index("c")
    s = lax.axis_index("s")
    wid = c * NS + s
    rows_per = N_ROWS // (NC * NS)
    base = wid * rows_per
    lane = lax.iota(jnp.int32, L)

    def in_copies(ci, p):
        row0 = base + ci * CHUNK
        return (
            pltpu.make_async_copy(
                z_hbm.at[pl.ds(row0, CHUNK), pl.ds(D_FEAT, NUM_INTERVALS)],
                vin.at[p], insem.at[p]),
            pltpu.make_async_copy(t_hbm.at[pl.ds(row0, CHUNK)], tin.at[p],
                                  insem.at[p]),
        )

    def out_copies(ci, p):
        row0 = base + ci * CHUNK
        dst = pl.ds(row0, CHUNK)
        return (
            pltpu.make_async_copy(dtbuf.at[p], dt_hbm.at[dst], outsem.at[p]),
            pltpu.make_async_copy(indbuf.at[p], ind_hbm.at[dst], outsem.at[p]),
            pltpu.make_async_copy(dtindbuf.at[p], dtind_hbm.at[dst],
                                  outsem.at[p]),
            pltpu.make_async_copy(tauindbuf.at[p], tauind_hbm.at[dst],
                                  outsem.at[p]),
            pltpu.make_async_copy(taunextbuf.at[p], taunext_hbm.at[dst],
                                  outsem.at[p]),
        )

    for cp in in_copies(0, 0):
        cp.start()
    for cp in in_copies(1, 1):
        cp.start()

    def chunk_body(ci, carry):
        p = jnp.bitwise_and(ci, 1)
        for cp in in_copies(ci, p):
            cp.wait()

        @pl.when(ci >= 2)
        def _():
            for cp in out_copies(ci - 2, p):
                cp.wait()

        def group_body(g, gcarry):
            ind_acc = jnp.zeros((L,), jnp.int32)
            t16 = tin[p, pl.ds(g * L, L)]
            for j in range(L):
                r = g * L + j
                tj = t16[j]
                e = []
                cume = []
                for i in range(NV):
                    ei = jnp.exp(vin[p, r, pl.ds(i * L, L)])
                    e.append(ei)
                    cume.append(plsc.cumsum(ei))
                prefix = []
                tot = np.float32(0.0)
                for i in range(NV):
                    prefix.append(tot)
                    tot = tot + cume[i][L - 1]
                inv = 1.0 / jnp.broadcast_to(tot, (L,))
                cnt = jnp.zeros((L,), jnp.int32)
                for i in range(NV):
                    dtbuf[p, r, pl.ds(i * L, L)] = e[i] * inv
                    taui = (cume[i] + prefix[i]) * inv
                    taubuf[j, pl.ds(i * L, L)] = taui
                    m = taui < tj
                    if i == NV - 1:
                        m = m & (lane < L - 1)
                    cnt = cnt + m.astype(jnp.int32)
                indj = jnp.sum(cnt)
                ind_acc = ind_acc + jnp.where(lane == j, indj, 0)
            rows16 = g * L + lane
            p16 = jnp.broadcast_to(p, (L,))
            dtind16 = plsc.load_gather(dtbuf, [p16, rows16, ind_acc])
            taunext16 = plsc.load_gather(taubuf, [lane, ind_acc])
            indbuf[p, pl.ds(g * L, L)] = ind_acc
            dtindbuf[p, pl.ds(g * L, L)] = dtind16
            taunextbuf[p, pl.ds(g * L, L)] = taunext16
            tauindbuf[p, pl.ds(g * L, L)] = taunext16 - dtind16
            return gcarry

        lax.fori_loop(0, CHUNK // L, group_body, 0)

        for cp in out_copies(ci, p):
            cp.start()

        @pl.when(ci + 2 < NCHUNKS)
        def _():
            for cp in in_copies(ci + 2, p):
                cp.start()

        return carry

    lax.fori_loop(0, NCHUNKS, chunk_body, 0)

    for cp in out_copies(NCHUNKS - 2, 0):
        cp.wait()
    for cp in out_copies(NCHUNKS - 1, 1):
        cp.wait()


@jax.jit
def kernel(t, z):
    n = t.shape[0]
    mesh = plsc.VectorSubcoreMesh(core_axis_name="c", subcore_axis_name="s")
    out_type = (
        jax.ShapeDtypeStruct((n,), jnp.int32),
        jax.ShapeDtypeStruct((n, NUM_INTERVALS), jnp.float32),
        jax.ShapeDtypeStruct((n,), jnp.float32),
        jax.ShapeDtypeStruct((n,), jnp.float32),
        jax.ShapeDtypeStruct((n,), jnp.float32),
    )
    scratch = [
        pltpu.VMEM((2, CHUNK, NUM_INTERVALS), jnp.float32),
        pltpu.VMEM((2, CHUNK), jnp.float32),
        pltpu.VMEM((2, CHUNK, NUM_INTERVALS), jnp.float32),
        pltpu.VMEM((L, NUM_INTERVALS), jnp.float32),
        pltpu.VMEM((2, CHUNK), jnp.int32),
        pltpu.VMEM((2, CHUNK), jnp.float32),
        pltpu.VMEM((2, CHUNK), jnp.float32),
        pltpu.VMEM((2, CHUNK), jnp.float32),
        pltpu.SemaphoreType.DMA((2,)),
        pltpu.SemaphoreType.DMA((2,)),
    ]
    ind, dt, dt_ind, tau_ind, tau_next = pl.kernel(
        _sc_body,
        out_type=out_type,
        mesh=mesh,
        scratch_types=scratch,
        compiler_params=pltpu.CompilerParams(needs_layout_passes=False),
    )(t, z)
    z0 = z[:, :D_FEAT]
    return (ind, dt, dt_ind, tau_ind, tau_next, z0)
